# baseline (speedup 1.0000x reference)
import math

import jax
import jax.numpy as jnp
from jax.experimental import pallas as pl
from jax.experimental.pallas import tpu as pltpu

_B = 4
_L = 1024
_W = 32
_INC = 768
_OUTC = 512
_KC = 4
_H = 8
_DH = 64
_T = 77
_E = 4


def _gelu_exact(x):
    return 0.5 * x * (1.0 + jax.lax.erf(x * (1.0 / math.sqrt(2.0))))


def _body(x_ref, tf_ref, dcw_ref, dcb_ref, wtaps_ref, cvb_ref, proj_ref,
          n1w_ref, n1b_ref, n2w_ref, n2b_ref, wqT_ref, wk_ref, wv_ref,
          woT_ref, wob_ref, gw_ref, gb_ref, ew1_ref, eb1_ref, ew2_ref,
          eb2_ref, out_ref, f1_s, pooled_s):
    b = pl.program_id(0)
    k = pl.program_id(1)

    xb = x_ref[0, 0]
    part = jnp.dot(dcw_ref[...], xb, preferred_element_type=jnp.float32)

    @pl.when(k == 0)
    def _():
        f1_s[...] = part + dcb_ref[...]

    @pl.when(k > 0)
    def _():
        f1_s[...] += part

    @pl.when(k == _KC - 1)
    def _():
        f1 = f1_s[...]

        lane = jax.lax.broadcasted_iota(jnp.int32, (1, _L), 1)
        p_ = lane // _W
        q_ = lane % _W
        acc = jnp.zeros((_OUTC, _L), jnp.float32)
        for a in range(3):
            for c in range(3):
                s = (a - 1) * _W + (c - 1)
                prod = jnp.dot(wtaps_ref[a * 3 + c], f1,
                               preferred_element_type=jnp.float32)
                shifted = jnp.roll(prod, -s, axis=1) if s != 0 else prod
                valid = ((q_ + (c - 1) >= 0) & (q_ + (c - 1) < _W)
                         & (p_ + (a - 1) >= 0) & (p_ + (a - 1) < _W))
                acc = acc + jnp.where(valid, shifted, 0.0)
        f2 = jnp.maximum(acc + cvb_ref[...], 0.0)

        m = jnp.mean(f2, axis=0, keepdims=True)
        v = jnp.mean((f2 - m) ** 2, axis=0, keepdims=True)
        f_ln = (f2 - m) / jnp.sqrt(v + 1e-5) * n1w_ref[...] + n1b_ref[...]

        tf = tf_ref[0]
        ctx = jnp.dot(tf, proj_ref[...], preferred_element_type=jnp.float32)
        cm = jnp.mean(ctx, axis=1, keepdims=True)
        cv = jnp.mean((ctx - cm) ** 2, axis=1, keepdims=True)
        ctxn = (ctx - cm) / jnp.sqrt(cv + 1e-5) * n2w_ref[...] + n2b_ref[...]

        krm = jnp.dot(ctxn, wk_ref[...], preferred_element_type=jnp.float32)
        vrm = jnp.dot(ctxn, wv_ref[...], preferred_element_type=jnp.float32)
        qcm = jnp.dot(wqT_ref[...], f_ln, preferred_element_type=jnp.float32)

        scale = 1.0 / math.sqrt(_DH)
        outs = []
        for h in range(_H):
            kh = krm[:, h * _DH:(h + 1) * _DH]
            qh = qcm[h * _DH:(h + 1) * _DH, :]
            simT = jnp.dot(kh, qh, preferred_element_type=jnp.float32) * scale
            mx = jnp.max(simT, axis=0, keepdims=True)
            ex = jnp.exp(simT - mx)
            attnT = ex / jnp.sum(ex, axis=0, keepdims=True)
            vh = vrm[:, h * _DH:(h + 1) * _DH]
            oh = jax.lax.dot_general(vh, attnT, (((0,), (0,)), ((), ())),
                                     preferred_element_type=jnp.float32)
            outs.append(oh)
        ocm = jnp.concatenate(outs, axis=0)
        o2 = jnp.dot(woT_ref[...], ocm,
                     preferred_element_type=jnp.float32) + wob_ref[...]
        fsum = f_ln + o2

        ones_row = jnp.ones((1, _L), jnp.float32)
        prow = jax.lax.dot_general(ones_row, fsum, (((1,), (1,)), ((), ())),
                                   preferred_element_type=jnp.float32) / _L
        pooled_s[pl.ds(b, 1), :] = prow

    @pl.when((b == _B - 1) & (k == _KC - 1))
    def _():
        pooled = pooled_s[...]
        glog = jnp.dot(pooled, gw_ref[...],
                       preferred_element_type=jnp.float32) + gb_ref[...]
        gmx = jnp.max(glog, axis=1, keepdims=True)
        ge = jnp.exp(glog - gmx)
        g = ge / jnp.sum(ge, axis=1, keepdims=True)

        eos = []
        for e in range(_E):
            hh = jnp.dot(pooled, ew1_ref[e],
                         preferred_element_type=jnp.float32) + eb1_ref[e]
            hh = _gelu_exact(hh)
            eo_e = jnp.dot(hh, ew2_ref[e],
                           preferred_element_type=jnp.float32) + eb2_ref[e]
            eos.append(eo_e)
        eo = jnp.concatenate(eos, axis=1)

        eidx = jax.lax.broadcasted_iota(jnp.int32, (_B, _E), 1)
        gmin = jnp.min(g, axis=1, keepdims=True)
        excl = jnp.max(jnp.where(g <= gmin, eidx, -1), axis=1, keepdims=True)
        keep = eidx != excl
        out_ref[...] = jnp.sum(jnp.where(keep, g * eo, 0.0), axis=1,
                               keepdims=True)


def kernel(x, text_features, dc_w, dc_b, conv_w, conv_b, proj, norm1_w,
           norm1_b, norm2_w, norm2_b, wq, wk, wv, wo, wo_b, gate_w, gate_b,
           e_w1, e_b1, e_w2, e_b2):
    B = x.shape[0]
    xr = x.reshape(B, _KC, _INC * 4 // _KC, _L)
    dcw = dc_w.reshape(_INC, _INC * 4)
    wtaps = conv_w.transpose(2, 3, 0, 1).reshape(9, _OUTC, _INC)

    grid = (B, _KC)

    def const(*block):
        return pl.BlockSpec(block, lambda b, k: tuple(0 for _ in block))

    in_specs = [
        pl.BlockSpec((1, 1, _INC * 4 // _KC, _L),
                     lambda b, k: (b, k, 0, 0)),
        pl.BlockSpec((1, _T, _INC), lambda b, k: (b, 0, 0)),
        pl.BlockSpec((_INC, _INC * 4 // _KC), lambda b, k: (0, k)),
        const(_INC, 1),
        const(9, _OUTC, _INC),
        const(_OUTC, 1),
        const(_INC, _OUTC),
        const(_OUTC, 1), const(_OUTC, 1),
        const(1, _OUTC), const(1, _OUTC),
        const(_OUTC, _OUTC),
        const(_OUTC, _OUTC),
        const(_OUTC, _OUTC),
        const(_OUTC, _OUTC),
        const(_OUTC, 1),
        const(_OUTC, _E),
        const(1, _E),
        const(_E, _OUTC, _OUTC),
        const(_E, 1, _OUTC),
        const(_E, _OUTC, 1),
        const(_E, 1, 1),
    ]

    pred = pl.pallas_call(
        _body,
        grid=grid,
        in_specs=in_specs,
        out_specs=pl.BlockSpec((_B, 1), lambda b, k: (0, 0)),
        out_shape=jax.ShapeDtypeStruct((_B, 1), jnp.float32),
        scratch_shapes=[
            pltpu.VMEM((_INC, _L), jnp.float32),
            pltpu.VMEM((_B, _OUTC), jnp.float32),
        ],
    )(xr, text_features, dcw, dc_b.reshape(_INC, 1), wtaps,
      conv_b.reshape(_OUTC, 1), proj, norm1_w.reshape(_OUTC, 1),
      norm1_b.reshape(_OUTC, 1), norm2_w.reshape(1, _OUTC),
      norm2_b.reshape(1, _OUTC), wq.T, wk, wv, wo.T, wo_b.reshape(_OUTC, 1),
      gate_w, gate_b.reshape(1, _E), e_w1, e_b1.reshape(_E, 1, _OUTC),
      e_w2, e_b2.reshape(_E, 1, 1))
    return pred


# speedup vs baseline: 1.0229x; 1.0229x over previous
import math

import jax
import jax.numpy as jnp
from jax.experimental import pallas as pl
from jax.experimental.pallas import tpu as pltpu

_B = 4
_L = 1024
_W = 32
_INC = 768
_OUTC = 512
_KC = 4
_H = 8
_DH = 64
_T = 77
_E = 4


def _gelu_exact(x):
    return 0.5 * x * (1.0 + jax.lax.erf(x * (1.0 / math.sqrt(2.0))))


def _body(x_ref, tf_ref, dcw_ref, dcb_ref, wtaps_ref, cvb_ref, proj_ref,
          n1w_ref, n1b_ref, n2w_ref, n2b_ref, wqT_ref, wk_ref, wv_ref,
          woT_ref, wob_ref, gw_ref, gb_ref, ew1_ref, eb1_ref, ew2_ref,
          eb2_ref, out_ref, f1_s, pooled_s):
    b = pl.program_id(0)
    k = pl.program_id(1)

    xb = x_ref[0, 0].astype(jnp.bfloat16)
    part = jnp.dot(dcw_ref[...], xb, preferred_element_type=jnp.float32)

    @pl.when(k == 0)
    def _():
        f1_s[...] = part + dcb_ref[...]

    @pl.when(k > 0)
    def _():
        f1_s[...] += part

    @pl.when(k == _KC - 1)
    def _():
        f1 = f1_s[...].astype(jnp.bfloat16)

        lane = jax.lax.broadcasted_iota(jnp.int32, (1, _L), 1)
        p_ = lane // _W
        q_ = lane % _W
        acc = jnp.zeros((_OUTC, _L), jnp.float32)
        for a in range(3):
            for c in range(3):
                s = (a - 1) * _W + (c - 1)
                prod = jnp.dot(wtaps_ref[a * 3 + c], f1,
                               preferred_element_type=jnp.float32)
                shifted = jnp.roll(prod, -s, axis=1) if s != 0 else prod
                valid = ((q_ + (c - 1) >= 0) & (q_ + (c - 1) < _W)
                         & (p_ + (a - 1) >= 0) & (p_ + (a - 1) < _W))
                acc = acc + jnp.where(valid, shifted, 0.0)
        f2 = jnp.maximum(acc + cvb_ref[...], 0.0)

        m = jnp.mean(f2, axis=0, keepdims=True)
        v = jnp.mean((f2 - m) ** 2, axis=0, keepdims=True)
        f_ln = (f2 - m) / jnp.sqrt(v + 1e-5) * n1w_ref[...] + n1b_ref[...]

        tf = tf_ref[0]
        ctx = jnp.dot(tf, proj_ref[...], preferred_element_type=jnp.float32)
        cm = jnp.mean(ctx, axis=1, keepdims=True)
        cv = jnp.mean((ctx - cm) ** 2, axis=1, keepdims=True)
        ctxn = (ctx - cm) / jnp.sqrt(cv + 1e-5) * n2w_ref[...] + n2b_ref[...]

        krm = jnp.dot(ctxn, wk_ref[...], preferred_element_type=jnp.float32)
        vrm = jnp.dot(ctxn, wv_ref[...], preferred_element_type=jnp.float32)
        qcm = jnp.dot(wqT_ref[...], f_ln, preferred_element_type=jnp.float32)

        scale = 1.0 / math.sqrt(_DH)
        outs = []
        for h in range(_H):
            kh = krm[:, h * _DH:(h + 1) * _DH]
            qh = qcm[h * _DH:(h + 1) * _DH, :]
            simT = jnp.dot(kh, qh, preferred_element_type=jnp.float32) * scale
            mx = jnp.max(simT, axis=0, keepdims=True)
            ex = jnp.exp(simT - mx)
            attnT = ex / jnp.sum(ex, axis=0, keepdims=True)
            vh = vrm[:, h * _DH:(h + 1) * _DH]
            oh = jax.lax.dot_general(vh, attnT, (((0,), (0,)), ((), ())),
                                     preferred_element_type=jnp.float32)
            outs.append(oh)
        ocm = jnp.concatenate(outs, axis=0)
        o2 = jnp.dot(woT_ref[...], ocm,
                     preferred_element_type=jnp.float32) + wob_ref[...]
        fsum = f_ln + o2

        ones_row = jnp.ones((1, _L), jnp.float32)
        prow = jax.lax.dot_general(ones_row, fsum, (((1,), (1,)), ((), ())),
                                   preferred_element_type=jnp.float32) / _L
        pooled_s[pl.ds(b, 1), :] = prow

    @pl.when((b == _B - 1) & (k == _KC - 1))
    def _():
        pooled = pooled_s[...]
        glog = jnp.dot(pooled, gw_ref[...],
                       preferred_element_type=jnp.float32) + gb_ref[...]
        gmx = jnp.max(glog, axis=1, keepdims=True)
        ge = jnp.exp(glog - gmx)
        g = ge / jnp.sum(ge, axis=1, keepdims=True)

        eos = []
        for e in range(_E):
            hh = jnp.dot(pooled, ew1_ref[e],
                         preferred_element_type=jnp.float32) + eb1_ref[e]
            hh = _gelu_exact(hh)
            eo_e = jnp.dot(hh, ew2_ref[e],
                           preferred_element_type=jnp.float32) + eb2_ref[e]
            eos.append(eo_e)
        eo = jnp.concatenate(eos, axis=1)

        eidx = jax.lax.broadcasted_iota(jnp.int32, (_B, _E), 1)
        gmin = jnp.min(g, axis=1, keepdims=True)
        excl = jnp.max(jnp.where(g <= gmin, eidx, -1), axis=1, keepdims=True)
        keep = eidx != excl
        out_ref[...] = jnp.sum(jnp.where(keep, g * eo, 0.0), axis=1,
                               keepdims=True)


def kernel(x, text_features, dc_w, dc_b, conv_w, conv_b, proj, norm1_w,
           norm1_b, norm2_w, norm2_b, wq, wk, wv, wo, wo_b, gate_w, gate_b,
           e_w1, e_b1, e_w2, e_b2):
    B = x.shape[0]
    xr = x.reshape(B, _KC, _INC * 4 // _KC, _L)
    dcw = dc_w.reshape(_INC, _INC * 4).astype(jnp.bfloat16)
    wtaps = conv_w.transpose(2, 3, 0, 1).reshape(9, _OUTC, _INC).astype(
        jnp.bfloat16)

    grid = (B, _KC)

    def const(*block):
        return pl.BlockSpec(block, lambda b, k: tuple(0 for _ in block))

    in_specs = [
        pl.BlockSpec((1, 1, _INC * 4 // _KC, _L),
                     lambda b, k: (b, k, 0, 0)),
        pl.BlockSpec((1, _T, _INC), lambda b, k: (b, 0, 0)),
        pl.BlockSpec((_INC, _INC * 4 // _KC), lambda b, k: (0, k)),
        const(_INC, 1),
        const(9, _OUTC, _INC),
        const(_OUTC, 1),
        const(_INC, _OUTC),
        const(_OUTC, 1), const(_OUTC, 1),
        const(1, _OUTC), const(1, _OUTC),
        const(_OUTC, _OUTC),
        const(_OUTC, _OUTC),
        const(_OUTC, _OUTC),
        const(_OUTC, _OUTC),
        const(_OUTC, 1),
        const(_OUTC, _E),
        const(1, _E),
        const(_E, _OUTC, _OUTC),
        const(_E, 1, _OUTC),
        const(_E, _OUTC, 1),
        const(_E, 1, 1),
    ]

    pred = pl.pallas_call(
        _body,
        grid=grid,
        in_specs=in_specs,
        out_specs=pl.BlockSpec((_B, 1), lambda b, k: (0, 0)),
        out_shape=jax.ShapeDtypeStruct((_B, 1), jnp.float32),
        scratch_shapes=[
            pltpu.VMEM((_INC, _L), jnp.float32),
            pltpu.VMEM((_B, _OUTC), jnp.float32),
        ],
    )(xr, text_features, dcw, dc_b.reshape(_INC, 1), wtaps,
      conv_b.reshape(_OUTC, 1), proj, norm1_w.reshape(_OUTC, 1),
      norm1_b.reshape(_OUTC, 1), norm2_w.reshape(1, _OUTC),
      norm2_b.reshape(1, _OUTC), wq.T, wk, wv, wo.T, wo_b.reshape(_OUTC, 1),
      gate_w, gate_b.reshape(1, _E), e_w1, e_b1.reshape(_E, 1, _OUTC),
      e_w2, e_b2.reshape(_E, 1, 1))
    return pred
